# baseline (speedup 1.0000x reference)
import jax
import jax.numpy as jnp
from jax.experimental import pallas as pl
from jax.experimental.pallas import tpu as pltpu

B, S, D = 1, 2048, 768
E, K, I = 8, 2, 256
SI = 1024
EI = E * I

BT = 256


def _moe_body(x_ref, rw_ref, gw_hbm, uw_hbm, dw_hbm, sgw_hbm, suw_hbm,
              sdw_hbm, out_ref, logits_ref,
              gw_v, uw_v, dw_v, sgw_v, suw_v, sdw_v,
              s_gw, s_uw, s_dw, s_sgw, s_suw, s_sdw):
    i = pl.program_id(0)

    @pl.when(i == 0)
    def _start_weight_dma():
        pltpu.make_async_copy(gw_hbm, gw_v, s_gw).start()
        pltpu.make_async_copy(uw_hbm, uw_v, s_uw).start()
        pltpu.make_async_copy(dw_hbm, dw_v, s_dw).start()
        pltpu.make_async_copy(sgw_hbm, sgw_v, s_sgw).start()
        pltpu.make_async_copy(suw_hbm, suw_v, s_suw).start()
        pltpu.make_async_copy(sdw_hbm, sdw_v, s_sdw).start()

    x = x_ref[...]

    logits = jax.lax.dot_general(
        x, rw_ref[...], (((1,), (1,)), ((), ())),
        preferred_element_type=jnp.float32)
    logits_ref[...] = logits
    xb = x.astype(jnp.bfloat16)

    lane = jax.lax.broadcasted_iota(jnp.int32, logits.shape, 1)
    big = jnp.int32(E + 1)
    m1 = jnp.max(logits, axis=1, keepdims=True)
    i1 = jnp.min(jnp.where(logits == m1, lane, big), axis=1, keepdims=True)
    masked = jnp.where(lane == i1, -jnp.inf, logits)
    m2 = jnp.max(masked, axis=1, keepdims=True)
    i2 = jnp.min(jnp.where(masked == m2, lane, big), axis=1, keepdims=True)
    e2 = jnp.exp(m2 - m1)
    w1 = 1.0 / (1.0 + e2)
    w2 = e2 * w1

    @pl.when(i == 0)
    def _wait_gate_up():
        pltpu.make_async_copy(gw_hbm, gw_v, s_gw).wait()
        pltpu.make_async_copy(uw_hbm, uw_v, s_uw).wait()

    g = jax.lax.dot_general(
        xb, gw_v[...], (((1,), (1,)), ((), ())),
        preferred_element_type=jnp.float32)
    u = jax.lax.dot_general(
        xb, uw_v[...], (((1,), (1,)), ((), ())),
        preferred_element_type=jnp.float32)
    echunk = jax.lax.broadcasted_iota(jnp.int32, g.shape, 1) // I
    c = jnp.where(echunk == i1, w1, 0.0) + jnp.where(echunk == i2, w2, 0.0)
    h = (g * jax.nn.sigmoid(g) * u * c).astype(jnp.bfloat16)

    @pl.when(i == 0)
    def _wait_rest():
        pltpu.make_async_copy(dw_hbm, dw_v, s_dw).wait()
        pltpu.make_async_copy(sgw_hbm, sgw_v, s_sgw).wait()
        pltpu.make_async_copy(suw_hbm, suw_v, s_suw).wait()
        pltpu.make_async_copy(sdw_hbm, sdw_v, s_sdw).wait()

    acc = jnp.zeros((x.shape[0], D), dtype=jnp.float32)
    for e in range(E):
        acc += jax.lax.dot_general(
            h[:, e * I:(e + 1) * I], dw_v[e], (((1,), (1,)), ((), ())),
            preferred_element_type=jnp.float32)

    sg = jax.lax.dot_general(
        xb, sgw_v[...], (((1,), (1,)), ((), ())),
        preferred_element_type=jnp.float32)
    su = jax.lax.dot_general(
        xb, suw_v[...], (((1,), (1,)), ((), ())),
        preferred_element_type=jnp.float32)
    sh = (sg * jax.nn.sigmoid(sg) * su).astype(jnp.bfloat16)
    sy = jax.lax.dot_general(
        sh, sdw_v[...], (((1,), (1,)), ((), ())),
        preferred_element_type=jnp.float32)
    out_ref[...] = acc + sy


@jax.jit
def _moe(x, router_w, gate_w, up_w, down_w, sgw, suw, sdw):
    t = x.shape[0]
    grid = (t // BT,)
    out, logits = pl.pallas_call(
        _moe_body,
        grid=grid,
        in_specs=[
            pl.BlockSpec((BT, D), lambda i: (i, 0)),
            pl.BlockSpec((E, D), lambda i: (0, 0)),
            pl.BlockSpec(memory_space=pl.ANY),
            pl.BlockSpec(memory_space=pl.ANY),
            pl.BlockSpec(memory_space=pl.ANY),
            pl.BlockSpec(memory_space=pl.ANY),
            pl.BlockSpec(memory_space=pl.ANY),
            pl.BlockSpec(memory_space=pl.ANY),
        ],
        out_specs=[
            pl.BlockSpec((BT, D), lambda i: (i, 0)),
            pl.BlockSpec((BT, E), lambda i: (i, 0)),
        ],
        out_shape=[
            jax.ShapeDtypeStruct((t, D), jnp.float32),
            jax.ShapeDtypeStruct((t, E), jnp.float32),
        ],
        scratch_shapes=[
            pltpu.VMEM((EI, D), jnp.float32),
            pltpu.VMEM((EI, D), jnp.float32),
            pltpu.VMEM((E, D, I), jnp.float32),
            pltpu.VMEM((SI, D), jnp.float32),
            pltpu.VMEM((SI, D), jnp.float32),
            pltpu.VMEM((D, SI), jnp.float32),
            pltpu.SemaphoreType.DMA,
            pltpu.SemaphoreType.DMA,
            pltpu.SemaphoreType.DMA,
            pltpu.SemaphoreType.DMA,
            pltpu.SemaphoreType.DMA,
            pltpu.SemaphoreType.DMA,
        ],
    )(x, router_w, gate_w.reshape(EI, D), up_w.reshape(EI, D), down_w,
      sgw, suw, sdw)
    return out, logits


def kernel(hidden_states, router_w, gate_w, up_w, down_w, shared_gate_w,
           shared_up_w, shared_down_w, training):
    b, s, d = hidden_states.shape
    x = hidden_states.reshape(b * s, d)
    out, logits = _moe(x, router_w, gate_w, up_w, down_w,
                       shared_gate_w[0], shared_up_w[0], shared_down_w[0])
    return out.reshape(b, s, d), logits


# speedup vs baseline: 1.1304x; 1.1304x over previous
import jax
import jax.numpy as jnp
from jax.experimental import pallas as pl
from jax.experimental.pallas import tpu as pltpu

B, S, D = 1, 2048, 768
E, K, I = 8, 2, 256
SI = 1024
EI = E * I

BT = 512


def _moe_body(x_ref, rw_ref, gw_ref, uw_ref, dw_ref, sgw_ref, suw_ref,
              sdw_ref, out_ref, logits_ref):
    x = x_ref[...]

    logits = jax.lax.dot_general(
        x, rw_ref[...], (((1,), (1,)), ((), ())),
        preferred_element_type=jnp.float32)
    logits_ref[...] = logits
    xb = x.astype(jnp.bfloat16)

    lane = jax.lax.broadcasted_iota(jnp.int32, logits.shape, 1)
    big = jnp.int32(E + 1)
    m1 = jnp.max(logits, axis=1, keepdims=True)
    i1 = jnp.min(jnp.where(logits == m1, lane, big), axis=1, keepdims=True)
    masked = jnp.where(lane == i1, -jnp.inf, logits)
    m2 = jnp.max(masked, axis=1, keepdims=True)
    i2 = jnp.min(jnp.where(masked == m2, lane, big), axis=1, keepdims=True)
    e2 = jnp.exp(m2 - m1)
    w1 = 1.0 / (1.0 + e2)
    w2 = e2 * w1

    g = jax.lax.dot_general(
        xb, gw_ref[...], (((1,), (1,)), ((), ())),
        preferred_element_type=jnp.float32)
    u = jax.lax.dot_general(
        xb, uw_ref[...], (((1,), (1,)), ((), ())),
        preferred_element_type=jnp.float32)
    echunk = jax.lax.broadcasted_iota(jnp.int32, g.shape, 1) // I
    c = jnp.where(echunk == i1, w1, 0.0) + jnp.where(echunk == i2, w2, 0.0)
    h = (g * jax.nn.sigmoid(g) * u * c).astype(jnp.bfloat16)

    acc = jnp.zeros((x.shape[0], D), dtype=jnp.float32)
    for e in range(E):
        acc += jax.lax.dot_general(
            h[:, e * I:(e + 1) * I], dw_ref[e], (((1,), (1,)), ((), ())),
            preferred_element_type=jnp.float32)

    sg = jax.lax.dot_general(
        xb, sgw_ref[...], (((1,), (1,)), ((), ())),
        preferred_element_type=jnp.float32)
    su = jax.lax.dot_general(
        xb, suw_ref[...], (((1,), (1,)), ((), ())),
        preferred_element_type=jnp.float32)
    sh = (sg * jax.nn.sigmoid(sg) * su).astype(jnp.bfloat16)
    sy = jax.lax.dot_general(
        sh, sdw_ref[...], (((1,), (1,)), ((), ())),
        preferred_element_type=jnp.float32)
    out_ref[...] = acc + sy


@jax.jit
def _moe(x, router_w, gate_w, up_w, down_w, sgw, suw, sdw):
    t = x.shape[0]
    grid = (t // BT,)
    out, logits = pl.pallas_call(
        _moe_body,
        grid=grid,
        in_specs=[
            pl.BlockSpec((BT, D), lambda i: (i, 0)),
            pl.BlockSpec((E, D), lambda i: (0, 0)),
            pl.BlockSpec((EI, D), lambda i: (0, 0)),
            pl.BlockSpec((EI, D), lambda i: (0, 0)),
            pl.BlockSpec((E, D, I), lambda i: (0, 0, 0)),
            pl.BlockSpec((SI, D), lambda i: (0, 0)),
            pl.BlockSpec((SI, D), lambda i: (0, 0)),
            pl.BlockSpec((D, SI), lambda i: (0, 0)),
        ],
        out_specs=[
            pl.BlockSpec((BT, D), lambda i: (i, 0)),
            pl.BlockSpec((BT, E), lambda i: (i, 0)),
        ],
        out_shape=[
            jax.ShapeDtypeStruct((t, D), jnp.float32),
            jax.ShapeDtypeStruct((t, E), jnp.float32),
        ],
    )(x, router_w, gate_w.reshape(EI, D), up_w.reshape(EI, D), down_w,
      sgw, suw, sdw)
    return out, logits


def kernel(hidden_states, router_w, gate_w, up_w, down_w, shared_gate_w,
           shared_up_w, shared_down_w, training):
    b, s, d = hidden_states.shape
    x = hidden_states.reshape(b * s, d)
    out, logits = _moe(x, router_w, gate_w, up_w, down_w,
                       shared_gate_w[0], shared_up_w[0], shared_down_w[0])
    return out.reshape(b, s, d), logits
